# baseline (speedup 1.0000x reference)
import jax
import jax.numpy as jnp
from jax import lax
from jax.experimental import pallas as pl
from jax.experimental.pallas import tpu as pltpu
from jax.experimental.pallas import tpu_sc as plsc

B, F, V, D_NAME, D_VAL = 16384, 100, 100, 16, 16
D_OUT = D_NAME + D_VAL
NC, NS = 2, 16
NW = NC * NS
ROWS_PER_W = B // NW
R = 16
CHUNKS = ROWS_PER_W // R


def _sc_body(fv_hbm, tbl_hbm, w_hbm, b_hbm, idx_hbm, out_hbm,
             idxv, namev, fvbuf, outbuf, wbuf, bbuf, sem):
    wid = lax.axis_index("s") * NC + lax.axis_index("c")
    base = wid * ROWS_PER_W

    pltpu.sync_copy(idx_hbm, idxv)
    pltpu.sync_copy(w_hbm, wbuf)
    pltpu.sync_copy(b_hbm, bbuf)
    pltpu.async_copy(tbl_hbm.at[idxv], namev, sem).wait()

    wv = wbuf[...]
    bv = bbuf[...]

    def fill_f(f, carry):
        nv = namev[f]
        for r in range(R):
            outbuf[r, f, pl.ds(0, D_NAME)] = nv
        return carry

    lax.fori_loop(0, F, fill_f, 0)

    def chunk_body(c, carry):
        row0 = base + c * R
        pltpu.sync_copy(fv_hbm.at[pl.ds(row0, R)], fvbuf)

        def row_body(r, rcarry):
            for f0 in (0, 16, 32, 48, 64, 80, F - D_VAL):
                fvv = fvbuf[r, pl.ds(f0, 16)]
                for j in range(16):
                    outbuf[r, f0 + j, pl.ds(D_NAME, D_VAL)] = fvv[j] * wv + bv
            return rcarry

        lax.fori_loop(0, R, row_body, 0)
        pltpu.sync_copy(outbuf, out_hbm.at[pl.ds(row0, R)])
        return carry

    lax.fori_loop(0, CHUNKS, chunk_body, 0)


@jax.jit
def kernel(feature_values, name_table, W, b, name_indices):
    w16 = W.reshape(D_VAL).astype(jnp.float32)
    b16 = b.astype(jnp.float32)
    mesh = plsc.VectorSubcoreMesh(
        core_axis_name="c", subcore_axis_name="s",
        num_cores=NC, num_subcores=NS)
    fn = pl.kernel(
        _sc_body,
        out_type=jax.ShapeDtypeStruct((B, F, D_OUT), jnp.float32),
        mesh=mesh,
        scratch_types=[
            pltpu.VMEM((F,), jnp.int32),
            pltpu.VMEM((F, D_NAME), jnp.float32),
            pltpu.VMEM((R, F), jnp.float32),
            pltpu.VMEM((R, F, D_OUT), jnp.float32),
            pltpu.VMEM((D_VAL,), jnp.float32),
            pltpu.VMEM((D_VAL,), jnp.float32),
            pltpu.SemaphoreType.DMA,
        ],
        compiler_params=pltpu.CompilerParams(use_tc_tiling_on_sc=False),
    )
    return fn(feature_values, name_table, w16, b16, name_indices)
